# baseline (speedup 1.0000x reference)
import functools

import jax
import jax.numpy as jnp
from jax import lax
from jax.experimental import pallas as pl
from jax.experimental.pallas import tpu as pltpu
from jax.experimental.pallas import tpu_sc as plsc

S = 2048
D = 768
R = 192
RP = 256
NI = 8
NP = 32
K = 3
NC = 3
NB = NC * NI
TB = 256
WRP = 128

SC_CORES = 2
SC_SUBCORES = 16
NW = SC_SUBCORES
TPW = S // NW
LANES = 16


def _router_body(x_ref, wr_ref, spt_ref):
    x = x_ref[...]
    scores = lax.dot_general(x, wr_ref[...], (((1,), (0,)), ((), ())),
                             preferred_element_type=jnp.float32)
    sp = scores[:, NI:NI + NP]
    spt_ref[...] = jnp.transpose(sp, (1, 0))


def _topk_sc_body(spt_hbm, idx_hbm, sbuf, ibuf):
    cid = lax.axis_index("c")
    sid = lax.axis_index("s")

    @pl.when(cid == 0)
    def _work():
        base = sid * TPW
        pltpu.sync_copy(spt_hbm.at[:, pl.ds(base, TPW)], sbuf)
        for g in range(TPW // LANES):
            lo = g * LANES
            neg = jnp.full((LANES,), -jnp.inf, jnp.float32)
            zero = jnp.zeros((LANES,), jnp.int32)
            m1, m2, m3 = neg, neg, neg
            i1, i2, i3 = zero, zero, zero
            for e in range(NP):
                v = sbuf[e, lo:lo + LANES]
                ev = jnp.full((LANES,), e, jnp.int32)
                gt1 = v > m1
                gt2 = v > m2
                gt3 = v > m3
                m3 = jnp.where(gt2, m2, jnp.where(gt3, v, m3))
                i3 = jnp.where(gt2, i2, jnp.where(gt3, ev, i3))
                m2 = jnp.where(gt1, m1, jnp.where(gt2, v, m2))
                i2 = jnp.where(gt1, i1, jnp.where(gt2, ev, i2))
                m1 = jnp.where(gt1, v, m1)
                i1 = jnp.where(gt1, ev, i1)
            ibuf[0, lo:lo + LANES] = i1
            ibuf[1, lo:lo + LANES] = i2
            ibuf[2, lo:lo + LANES] = i3
            for r in range(K, 8):
                ibuf[r, lo:lo + LANES] = zero
        pltpu.sync_copy(ibuf, idx_hbm.at[:, pl.ds(base, TPW)])


_topk_sc = functools.partial(
    pl.kernel,
    out_type=jax.ShapeDtypeStruct((8, S), jnp.int32),
    mesh=plsc.VectorSubcoreMesh(core_axis_name="c", subcore_axis_name="s"),
    scratch_types=[
        pltpu.VMEM((NP, TPW), jnp.float32),
        pltpu.VMEM((8, TPW), jnp.int32),
    ],
)(_topk_sc_body)


def _main_body(x_ref, wr_ref, in_ref, pn_ref, idx_ref,
               q_ref, k_ref, v_ref, inbf_ref):
    t = pl.program_id(0)

    @pl.when(t == 0)
    def _stage():
        for j in range(NB):
            inbf_ref[:, j * RP:(j + 1) * RP] = in_ref[j]

    x = x_ref[...]
    scores = lax.dot_general(x, wr_ref[...], (((1,), (0,)), ((), ())),
                             preferred_element_type=jnp.float32)
    si = scores[:, :NI]
    si = si - jnp.max(si, axis=-1, keepdims=True)
    e = jnp.exp(si)
    w = e / jnp.sum(e, axis=-1, keepdims=True)

    proj = lax.dot_general(x.astype(jnp.bfloat16), inbf_ref[...],
                           (((1,), (0,)), ((), ())),
                           preferred_element_type=jnp.float32)
    xrs = []
    for c in range(NC):
        xr = w[:, 0:1] * proj[:, c * NI * RP:c * NI * RP + RP]
        for n in range(1, NI):
            base = (c * NI + n) * RP
            xr = xr + w[:, n:n + 1] * proj[:, base:base + RP]
        xrs.append(xr[:, :R])

    pn_ns = []
    for c in range(NC):
        blk = pn_ref[c]
        nrm = lax.rsqrt(jnp.sum(blk * blk, axis=-1, keepdims=True) + 1e-8)
        pn_ns.append((blk * nrm)[:, :R])

    iota = lax.broadcasted_iota(jnp.int32, (TB, NP), 1)
    for kk in range(K):
        col = jnp.transpose(idx_ref[kk:kk + 1, :], (1, 0))
        ohf = (iota == col).astype(jnp.float32)
        for c in range(NC):
            sel = lax.dot_general(ohf, pn_ns[c], (((1,), (0,)), ((), ())),
                                  preferred_element_type=jnp.float32)
            vtx = jnp.sum(xrs[c] * sel, axis=-1, keepdims=True)
            xrs[c] = xrs[c] - 2.0 * sel * vtx

    q_ref[...] = xrs[0]
    k_ref[...] = xrs[1]
    v_ref[...] = xrs[2]


def kernel(x, Wi, Wp, q_in, q_pn, k_in, k_pn, v_in, v_pn):
    x2 = x.reshape(S, D)
    wr = jnp.concatenate([Wi.T, Wp.T], axis=1)
    wr = jnp.pad(wr, ((0, 0), (0, WRP - NI - NP)))
    instk = jnp.concatenate([q_in, k_in, v_in], axis=0)
    instk = jnp.pad(instk, ((0, 0), (0, 0), (0, RP - R)))
    instk = instk.astype(jnp.bfloat16)
    pnstk = jnp.stack([q_pn, k_pn, v_pn])
    pnstk = jnp.pad(pnstk, ((0, 0), (0, 0), (0, RP - R)))
    full = lambda shape: pl.BlockSpec(shape, lambda t: tuple(0 for _ in shape))

    spt = pl.pallas_call(
        _router_body,
        grid=(S // TB,),
        in_specs=[
            pl.BlockSpec((TB, D), lambda t: (t, 0)),
            full((D, WRP)),
        ],
        out_specs=pl.BlockSpec((NP, TB), lambda t: (0, t)),
        out_shape=jax.ShapeDtypeStruct((NP, S), jnp.float32),
    )(x2, wr)

    idx = _topk_sc(spt)

    q, k, v = pl.pallas_call(
        _main_body,
        grid=(S // TB,),
        in_specs=[
            pl.BlockSpec((TB, D), lambda t: (t, 0)),
            full((D, WRP)),
            full((NB, D, RP)),
            full((NC, NP, RP)),
            pl.BlockSpec((8, TB), lambda t: (0, t)),
        ],
        out_specs=[
            pl.BlockSpec((TB, R), lambda t: (t, 0)),
            pl.BlockSpec((TB, R), lambda t: (t, 0)),
            pl.BlockSpec((TB, R), lambda t: (t, 0)),
        ],
        out_shape=[jax.ShapeDtypeStruct((S, R), jnp.float32)] * 3,
        scratch_shapes=[pltpu.VMEM((D, NB * RP), jnp.bfloat16)],
    )(x2, wr, instk, pnstk, idx)
    return (q.reshape(1, S, R), k.reshape(1, S, R), v.reshape(1, S, R))
